# baseline (speedup 1.0000x reference)
import functools

import jax
import jax.numpy as jnp
from jax.experimental import pallas as pl

N = 10000
E = 320000
NM = 40000
EM = 160000
B = 256
LD = 50
LT = 16
D = 128
HID = 256
HEADS = 8
DH = 16
VOCAB = 10000


def _masked_mean_body(emb_ref, tok_ref, o_ref):
    emb = emb_ref[...]
    mask = (tok_ref[...] != 0).astype(emb.dtype)
    s = jnp.sum(emb * mask[..., None], axis=1)
    lens = jnp.sum(mask, axis=1)
    o_ref[...] = s / jnp.maximum(lens, 1.0)[:, None]


def masked_mean_pallas(emb, tok, block_rows):
    R, L, _ = emb.shape
    grid = (R // block_rows,)
    return pl.pallas_call(
        _masked_mean_body,
        grid=grid,
        in_specs=[
            pl.BlockSpec((block_rows, L, D), lambda i: (i, 0, 0)),
            pl.BlockSpec((block_rows, L), lambda i: (i, 0)),
        ],
        out_specs=pl.BlockSpec((block_rows, D), lambda i: (i, 0)),
        out_shape=jax.ShapeDtypeStruct((R, D), emb.dtype),
    )(emb, tok)


def _matmul_bias_body(x_ref, w_ref, b_ref, o_ref, *, relu_in):
    x = x_ref[...]
    if relu_in:
        x = jax.nn.relu(x + b_ref[...])
    o_ref[...] = jnp.dot(x, w_ref[...], preferred_element_type=jnp.float32)


def matmul_pallas(x, w, block_rows, bias_in=None):
    R, K = x.shape
    _, C = w.shape
    relu_in = bias_in is not None
    if bias_in is None:
        bias_in = jnp.zeros((1, K), x.dtype)
    else:
        bias_in = bias_in.reshape(1, K)
    grid = (R // block_rows,)
    return pl.pallas_call(
        functools.partial(_matmul_bias_body, relu_in=relu_in),
        grid=grid,
        in_specs=[
            pl.BlockSpec((block_rows, K), lambda i: (i, 0)),
            pl.BlockSpec((K, C), lambda i: (0, 0)),
            pl.BlockSpec((1, K), lambda i: (0, 0)),
        ],
        out_specs=pl.BlockSpec((block_rows, C), lambda i: (i, 0)),
        out_shape=jax.ShapeDtypeStruct((R, C), x.dtype),
    )(x, w, bias_in)


def _cosine_body(a_ref, b_ref, o_ref):
    a = a_ref[...]
    b = b_ref[...]
    num = jnp.sum(a * b, axis=-1)
    na = jnp.sqrt(jnp.sum(a * a, axis=-1))
    nb = jnp.sqrt(jnp.sum(b * b, axis=-1))
    o_ref[...] = num / (jnp.maximum(na, 1e-8) * jnp.maximum(nb, 1e-8))


def cosine_pallas(a, b):
    R, _ = a.shape
    return pl.pallas_call(
        _cosine_body,
        out_shape=jax.ShapeDtypeStruct((R,), a.dtype),
    )(a, b)


def _segment_softmax_sum(alpha, vals, seg, nseg):
    amax = jax.ops.segment_max(alpha, seg, num_segments=nseg)
    amax = jnp.where(jnp.isfinite(amax), amax, 0.0)
    ex = jnp.exp(alpha - amax[seg])
    den = jax.ops.segment_sum(ex, seg, num_segments=nseg)
    attn = ex / jnp.maximum(den[seg], 1e-16)
    if vals.ndim == attn.ndim + 1:
        attn = attn[..., None]
    return jax.ops.segment_sum(vals * attn, seg, num_segments=nseg)


def _gcn_norm(ei, n, dtype):
    row = jnp.concatenate([ei[0], jnp.arange(n, dtype=ei.dtype)])
    col = jnp.concatenate([ei[1], jnp.arange(n, dtype=ei.dtype)])
    deg = jax.ops.segment_sum(jnp.ones((row.shape[0],), dtype=dtype), col,
                              num_segments=n)
    dinv = jnp.where(deg > 0, jax.lax.rsqrt(jnp.maximum(deg, 1e-12)), 0.0)
    w = dinv[row] * dinv[col]
    return row, col, w


def kernel(x, edge_index, mini_x, mini_edge, mini_x_batch, batch, batch_desc,
           batch_lens, code_emb, code_emb2, desc_emb, Wq, bq, Wk, bk, Wv, bv,
           Wskip, bskip, W2, b2, W3, b3, Wg, bg):
    f32 = jnp.float32

    bd = batch_desc.astype(jnp.int32)
    pad = (-LD) % 16
    bd_p = jnp.pad(bd, ((0, 0), (0, pad)))
    de = desc_emb[bd_p]
    h_n = masked_mean_pallas(de, bd_p, B)

    xt = x.astype(jnp.int32)
    se = code_emb2[xt]
    stmt = masked_mean_pallas(se, xt, 400)

    mt = mini_x.astype(jnp.int32)
    me = code_emb[mt]
    mstmt = masked_mean_pallas(me, mt, 400)

    Wcat = jnp.concatenate([Wq, Wk, Wv, Wskip], axis=1)
    bcat = jnp.concatenate([bq, bk, bv, bskip], axis=0)
    proj = matmul_pallas(mstmt, Wcat, 2000) + bcat[None, :]
    q = proj[:, 0 * D:1 * D].reshape(NM, HEADS, DH)
    k = proj[:, 1 * D:2 * D].reshape(NM, HEADS, DH)
    v = proj[:, 2 * D:3 * D].reshape(NM, HEADS, DH)
    skip = proj[:, 3 * D:4 * D]

    src, dst = mini_edge[0], mini_edge[1]
    alpha = jnp.sum(q[dst] * k[src], axis=-1) * (1.0 / jnp.sqrt(float(DH)))
    agg = _segment_softmax_sum(alpha, v[src], dst, NM)
    mh = agg.reshape(NM, HEADS * DH) + skip

    Wg_p = jnp.pad(Wg, ((0, 0), (0, D - 1)))
    gate_m = matmul_pallas(mh, Wg_p, 2000)[:, 0] + bg[0]
    mfr = _segment_softmax_sum(gate_m, mh, mini_x_batch, N)
    mfr = (mfr + stmt) * 0.5

    row, col, wn = _gcn_norm(edge_index, N, f32)

    hx1 = matmul_pallas(mfr, W2, 2000)
    s1 = jax.ops.segment_sum(hx1[row] * wn[:, None], col, num_segments=N)
    hx2 = matmul_pallas(s1, W3, 2000, bias_in=b2)
    s2 = jax.ops.segment_sum(hx2[row] * wn[:, None], col, num_segments=N)
    h2 = s2 + b3[None, :]

    gate_h = matmul_pallas(h2, Wg_p, 2000)[:, 0] + bg[0]
    fr = _segment_softmax_sum(gate_h, h2, batch, B)

    return cosine_pallas(fr, h_n)


# speedup vs baseline: 1.1154x; 1.1154x over previous
import functools

import jax
import jax.numpy as jnp
from jax.experimental import pallas as pl

N = 10000
E = 320000
NM = 40000
EM = 160000
B = 256
LD = 50
LT = 16
D = 128
HID = 256
HEADS = 8
DH = 16
VOCAB = 10000


def _masked_mean_body(emb_ref, tok_ref, o_ref):
    emb = emb_ref[...]
    mask = (tok_ref[...] != 0).astype(emb.dtype)
    s = jnp.sum(emb * mask[..., None], axis=1)
    lens = jnp.sum(mask, axis=1)
    o_ref[...] = s / jnp.maximum(lens, 1.0)[:, None]


def masked_mean_pallas(emb, tok, block_rows):
    R, L, _ = emb.shape
    grid = (R // block_rows,)
    return pl.pallas_call(
        _masked_mean_body,
        grid=grid,
        in_specs=[
            pl.BlockSpec((block_rows, L, D), lambda i: (i, 0, 0)),
            pl.BlockSpec((block_rows, L), lambda i: (i, 0)),
        ],
        out_specs=pl.BlockSpec((block_rows, D), lambda i: (i, 0)),
        out_shape=jax.ShapeDtypeStruct((R, D), emb.dtype),
    )(emb, tok)


def _matmul_bias_body(x_ref, w_ref, b_ref, o_ref, *, relu_in):
    x = x_ref[...]
    if relu_in:
        x = jax.nn.relu(x + b_ref[...])
    o_ref[...] = jnp.dot(x, w_ref[...], preferred_element_type=jnp.float32)


def matmul_pallas(x, w, block_rows, bias_in=None):
    R, K = x.shape
    _, C = w.shape
    relu_in = bias_in is not None
    if bias_in is None:
        bias_in = jnp.zeros((1, K), x.dtype)
    else:
        bias_in = bias_in.reshape(1, K)
    grid = (R // block_rows,)
    return pl.pallas_call(
        functools.partial(_matmul_bias_body, relu_in=relu_in),
        grid=grid,
        in_specs=[
            pl.BlockSpec((block_rows, K), lambda i: (i, 0)),
            pl.BlockSpec((K, C), lambda i: (0, 0)),
            pl.BlockSpec((1, K), lambda i: (0, 0)),
        ],
        out_specs=pl.BlockSpec((block_rows, C), lambda i: (i, 0)),
        out_shape=jax.ShapeDtypeStruct((R, C), x.dtype),
    )(x, w, bias_in)


def _cosine_body(a_ref, b_ref, o_ref):
    a = a_ref[...]
    b = b_ref[...]
    num = jnp.sum(a * b, axis=-1)
    na = jnp.sqrt(jnp.sum(a * a, axis=-1))
    nb = jnp.sqrt(jnp.sum(b * b, axis=-1))
    o_ref[...] = num / (jnp.maximum(na, 1e-8) * jnp.maximum(nb, 1e-8))


def cosine_pallas(a, b):
    R, _ = a.shape
    return pl.pallas_call(
        _cosine_body,
        out_shape=jax.ShapeDtypeStruct((R,), a.dtype),
    )(a, b)


def _segment_softmax_sum(alpha, vals, seg, nseg):
    amax = jax.ops.segment_max(alpha, seg, num_segments=nseg)
    amax = jnp.where(jnp.isfinite(amax), amax, 0.0)
    ex = jnp.exp(alpha - amax[seg])
    den = jax.ops.segment_sum(ex, seg, num_segments=nseg)
    attn = ex / jnp.maximum(den[seg], 1e-16)
    if vals.ndim == attn.ndim + 1:
        attn = attn[..., None]
    return jax.ops.segment_sum(vals * attn, seg, num_segments=nseg)


def _gcn_norm(ei, n, dtype):
    row = jnp.concatenate([ei[0], jnp.arange(n, dtype=ei.dtype)])
    col = jnp.concatenate([ei[1], jnp.arange(n, dtype=ei.dtype)])
    deg = jax.ops.segment_sum(jnp.ones((row.shape[0],), dtype=dtype), col,
                              num_segments=n)
    dinv = jnp.where(deg > 0, jax.lax.rsqrt(jnp.maximum(deg, 1e-12)), 0.0)
    return row, col, dinv


def kernel(x, edge_index, mini_x, mini_edge, mini_x_batch, batch, batch_desc,
           batch_lens, code_emb, code_emb2, desc_emb, Wq, bq, Wk, bk, Wv, bv,
           Wskip, bskip, W2, b2, W3, b3, Wg, bg):
    f32 = jnp.float32

    bd = batch_desc.astype(jnp.int32)
    pad = (-LD) % 16
    bd_p = jnp.pad(bd, ((0, 0), (0, pad)))
    de = desc_emb[bd_p]
    h_n = masked_mean_pallas(de, bd_p, B)

    xt = x.astype(jnp.int32)
    se = code_emb2[xt]
    stmt = masked_mean_pallas(se, xt, 400)

    mt = mini_x.astype(jnp.int32)
    me = code_emb[mt]
    mstmt = masked_mean_pallas(me, mt, 400)

    Wcat = jnp.concatenate([Wq, Wk, Wv, Wskip], axis=1)
    bcat = jnp.concatenate([bq, bk, bv, bskip], axis=0)
    proj = matmul_pallas(mstmt, Wcat, 2000) + bcat[None, :]
    q = proj[:, 0 * D:1 * D].reshape(NM, HEADS, DH)
    k = proj[:, 1 * D:2 * D].reshape(NM, HEADS, DH)
    v = proj[:, 2 * D:3 * D].reshape(NM, HEADS, DH)
    skip = proj[:, 3 * D:4 * D]

    src, dst = mini_edge[0], mini_edge[1]
    alpha = jnp.sum(q[dst] * k[src], axis=-1) * (1.0 / jnp.sqrt(float(DH)))
    agg = _segment_softmax_sum(alpha, v[src], dst, NM)
    mh = agg.reshape(NM, HEADS * DH) + skip

    Wg_p = jnp.pad(Wg, ((0, 0), (0, D - 1)))
    gate_m = matmul_pallas(mh, Wg_p, 2000)[:, 0] + bg[0]
    mfr = _segment_softmax_sum(gate_m, mh, mini_x_batch, N)
    mfr = (mfr + stmt) * 0.5

    row, col, dinv = _gcn_norm(edge_index, N, f32)
    dinv_c = dinv[:, None]

    agg1 = dinv_c * jax.ops.segment_sum((mfr * dinv_c)[row], col,
                                        num_segments=N)
    s1 = matmul_pallas(agg1, W2, 2000)
    hx2 = matmul_pallas(s1, W3, 2000, bias_in=b2)
    s2 = dinv_c * jax.ops.segment_sum((hx2 * dinv_c)[row], col,
                                      num_segments=N)
    h2 = s2 + b3[None, :]

    gate_h = matmul_pallas(h2, Wg_p, 2000)[:, 0] + bg[0]
    fr = _segment_softmax_sum(gate_h, h2, batch, B)

    return cosine_pallas(fr, h_n)


# speedup vs baseline: 1.7523x; 1.5710x over previous
import functools

import jax
import jax.numpy as jnp
from jax.experimental import pallas as pl

N = 10000
E = 320000
NM = 40000
EM = 160000
B = 256
LD = 50
LT = 16
D = 128
HID = 256
HEADS = 8
DH = 16
VOCAB = 10000


def _masked_mean_body(emb_ref, tok_ref, o_ref):
    emb = emb_ref[...]
    mask = (tok_ref[...] != 0).astype(emb.dtype)
    s = jnp.sum(emb * mask[..., None], axis=1)
    lens = jnp.sum(mask, axis=1)
    o_ref[...] = s / jnp.maximum(lens, 1.0)[:, None]


def masked_mean_pallas(emb, tok, block_rows):
    R, L, _ = emb.shape
    grid = (R // block_rows,)
    return pl.pallas_call(
        _masked_mean_body,
        grid=grid,
        in_specs=[
            pl.BlockSpec((block_rows, L, D), lambda i: (i, 0, 0)),
            pl.BlockSpec((block_rows, L), lambda i: (i, 0)),
        ],
        out_specs=pl.BlockSpec((block_rows, D), lambda i: (i, 0)),
        out_shape=jax.ShapeDtypeStruct((R, D), emb.dtype),
    )(emb, tok)


def _matmul_bias_body(x_ref, w_ref, b_ref, o_ref, *, relu_in):
    x = x_ref[...]
    if relu_in:
        x = jax.nn.relu(x + b_ref[...])
    o_ref[...] = jnp.dot(x, w_ref[...], preferred_element_type=jnp.float32)


def matmul_pallas(x, w, block_rows, bias_in=None):
    R, K = x.shape
    _, C = w.shape
    relu_in = bias_in is not None
    if bias_in is None:
        bias_in = jnp.zeros((1, K), x.dtype)
    else:
        bias_in = bias_in.reshape(1, K)
    grid = (R // block_rows,)
    return pl.pallas_call(
        functools.partial(_matmul_bias_body, relu_in=relu_in),
        grid=grid,
        in_specs=[
            pl.BlockSpec((block_rows, K), lambda i: (i, 0)),
            pl.BlockSpec((K, C), lambda i: (0, 0)),
            pl.BlockSpec((1, K), lambda i: (0, 0)),
        ],
        out_specs=pl.BlockSpec((block_rows, C), lambda i: (i, 0)),
        out_shape=jax.ShapeDtypeStruct((R, C), x.dtype),
    )(x, w, bias_in)


def _cosine_body(a_ref, b_ref, o_ref):
    a = a_ref[...]
    b = b_ref[...]
    num = jnp.sum(a * b, axis=-1)
    na = jnp.sqrt(jnp.sum(a * a, axis=-1))
    nb = jnp.sqrt(jnp.sum(b * b, axis=-1))
    o_ref[...] = num / (jnp.maximum(na, 1e-8) * jnp.maximum(nb, 1e-8))


def cosine_pallas(a, b):
    R, _ = a.shape
    return pl.pallas_call(
        _cosine_body,
        out_shape=jax.ShapeDtypeStruct((R,), a.dtype),
    )(a, b)


def _segment_softmax_sum(alpha, vals, seg, nseg):
    amax = jax.ops.segment_max(alpha, seg, num_segments=nseg)
    amax = jnp.where(jnp.isfinite(amax), amax, 0.0)
    ex = jnp.exp(alpha - amax[seg])
    den = jax.ops.segment_sum(ex, seg, num_segments=nseg)
    attn = ex / jnp.maximum(den[seg], 1e-16)
    if vals.ndim == attn.ndim + 1:
        attn = attn[..., None]
    return jax.ops.segment_sum(vals * attn, seg, num_segments=nseg)


def _gcn_norm(ei, n, dtype):
    row = jnp.concatenate([ei[0], jnp.arange(n, dtype=ei.dtype)])
    col = jnp.concatenate([ei[1], jnp.arange(n, dtype=ei.dtype)])
    deg = jax.ops.segment_sum(jnp.ones((row.shape[0],), dtype=dtype), col,
                              num_segments=n)
    dinv = jnp.where(deg > 0, jax.lax.rsqrt(jnp.maximum(deg, 1e-12)), 0.0)
    return row, col, dinv


def kernel(x, edge_index, mini_x, mini_edge, mini_x_batch, batch, batch_desc,
           batch_lens, code_emb, code_emb2, desc_emb, Wq, bq, Wk, bk, Wv, bv,
           Wskip, bskip, W2, b2, W3, b3, Wg, bg):
    f32 = jnp.float32

    bd = batch_desc.astype(jnp.int32)
    pad = (-LD) % 16
    bd_p = jnp.pad(bd, ((0, 0), (0, pad)))
    de = desc_emb[bd_p]
    h_n = masked_mean_pallas(de, bd_p, B)

    xt = x.astype(jnp.int32)
    se = code_emb2[xt]
    stmt = masked_mean_pallas(se, xt, 400)

    mt = mini_x.astype(jnp.int32)
    me = code_emb[mt]
    mstmt = masked_mean_pallas(me, mt, 400)

    Wcat = jnp.concatenate([Wq, Wk, Wv, Wskip], axis=1)
    bcat = jnp.concatenate([bq, bk, bv, bskip], axis=0)
    proj = matmul_pallas(mstmt, Wcat, 2000) + bcat[None, :]
    q = proj[:, 0 * D:1 * D]
    kv = proj[:, 1 * D:3 * D]
    skip = proj[:, 3 * D:4 * D]

    src, dst = mini_edge[0], mini_edge[1]
    kv_s = kv[src]
    k_s = kv_s[:, :D].reshape(EM, HEADS, DH)
    v_s = kv_s[:, D:].reshape(EM, HEADS, DH)
    q_d = q[dst].reshape(EM, HEADS, DH)
    alpha = jnp.sum(q_d * k_s, axis=-1) * (1.0 / jnp.sqrt(float(DH)))
    agg = _segment_softmax_sum(alpha, v_s, dst, NM)
    mh = agg.reshape(NM, HEADS * DH) + skip

    Wg_p = jnp.pad(Wg, ((0, 0), (0, D - 1)))
    gate_m = matmul_pallas(mh, Wg_p, 2000)[:, 0] + bg[0]
    mfr = _segment_softmax_sum(gate_m, mh, mini_x_batch, N)
    mfr = (mfr + stmt) * 0.5

    row, col, dinv = _gcn_norm(edge_index, N, f32)
    dinv_c = dinv[:, None]

    agg1 = dinv_c * jax.ops.segment_sum((mfr * dinv_c)[row], col,
                                        num_segments=N)
    s1 = matmul_pallas(agg1, W2, 2000)
    hx2 = matmul_pallas(s1, W3, 2000, bias_in=b2)
    s2 = dinv_c * jax.ops.segment_sum((hx2 * dinv_c)[row], col,
                                      num_segments=N)
    h2 = s2 + b3[None, :]

    gate_h = matmul_pallas(h2, Wg_p, 2000)[:, 0] + bg[0]
    fr = _segment_softmax_sum(gate_h, h2, batch, B)

    return cosine_pallas(fr, h_n)


# speedup vs baseline: 4.1865x; 2.3892x over previous
import functools

import jax
import jax.numpy as jnp
from jax.experimental import pallas as pl

N = 10000
E = 320000
NM = 40000
EM = 160000
B = 256
LD = 50
LT = 16
D = 128
HID = 256
HEADS = 8
DH = 16
VOCAB = 10000


def _masked_mean_body(emb_ref, tok_ref, o_ref):
    emb = emb_ref[...]
    mask = (tok_ref[...] != 0).astype(emb.dtype)
    s = jnp.sum(emb * mask[..., None], axis=1)
    lens = jnp.sum(mask, axis=1)
    o_ref[...] = s / jnp.maximum(lens, 1.0)[:, None]


def masked_mean_pallas(emb, tok, block_rows):
    R, L, _ = emb.shape
    grid = (R // block_rows,)
    return pl.pallas_call(
        _masked_mean_body,
        grid=grid,
        in_specs=[
            pl.BlockSpec((block_rows, L, D), lambda i: (i, 0, 0)),
            pl.BlockSpec((block_rows, L), lambda i: (i, 0)),
        ],
        out_specs=pl.BlockSpec((block_rows, D), lambda i: (i, 0)),
        out_shape=jax.ShapeDtypeStruct((R, D), emb.dtype),
    )(emb, tok)


def _matmul_bias_body(x_ref, w_ref, b_ref, o_ref, *, relu_in):
    x = x_ref[...]
    if relu_in:
        x = jax.nn.relu(x + b_ref[...])
    o_ref[...] = jnp.dot(x, w_ref[...], preferred_element_type=jnp.float32)


def matmul_pallas(x, w, block_rows, bias_in=None):
    R, K = x.shape
    _, C = w.shape
    relu_in = bias_in is not None
    if bias_in is None:
        bias_in = jnp.zeros((1, K), x.dtype)
    else:
        bias_in = bias_in.reshape(1, K)
    grid = (R // block_rows,)
    return pl.pallas_call(
        functools.partial(_matmul_bias_body, relu_in=relu_in),
        grid=grid,
        in_specs=[
            pl.BlockSpec((block_rows, K), lambda i: (i, 0)),
            pl.BlockSpec((K, C), lambda i: (0, 0)),
            pl.BlockSpec((1, K), lambda i: (0, 0)),
        ],
        out_specs=pl.BlockSpec((block_rows, C), lambda i: (i, 0)),
        out_shape=jax.ShapeDtypeStruct((R, C), x.dtype),
    )(x, w, bias_in)


def _cosine_body(a_ref, b_ref, o_ref):
    a = a_ref[...]
    b = b_ref[...]
    num = jnp.sum(a * b, axis=-1)
    na = jnp.sqrt(jnp.sum(a * a, axis=-1))
    nb = jnp.sqrt(jnp.sum(b * b, axis=-1))
    o_ref[...] = num / (jnp.maximum(na, 1e-8) * jnp.maximum(nb, 1e-8))


def cosine_pallas(a, b):
    R, _ = a.shape
    return pl.pallas_call(
        _cosine_body,
        out_shape=jax.ShapeDtypeStruct((R,), a.dtype),
    )(a, b)


def _segment_softmax_sum(alpha, vals, seg, nseg):
    amax = jax.ops.segment_max(alpha, seg, num_segments=nseg)
    amax = jnp.where(jnp.isfinite(amax), amax, 0.0)
    ex = jnp.exp(alpha - amax[seg])
    den = jax.ops.segment_sum(ex, seg, num_segments=nseg)
    attn = ex / jnp.maximum(den[seg], 1e-16)
    if vals.ndim == attn.ndim + 1:
        attn = attn[..., None]
    return jax.ops.segment_sum(vals * attn, seg, num_segments=nseg)


def _gcn_norm(ei, n, dtype):
    row = jnp.concatenate([ei[0], jnp.arange(n, dtype=ei.dtype)])
    col = jnp.concatenate([ei[1], jnp.arange(n, dtype=ei.dtype)])
    deg = jax.ops.segment_sum(jnp.ones((row.shape[0],), dtype=dtype), col,
                              num_segments=n)
    dinv = jnp.where(deg > 0, jax.lax.rsqrt(jnp.maximum(deg, 1e-12)), 0.0)
    return row, col, dinv


def kernel(x, edge_index, mini_x, mini_edge, mini_x_batch, batch, batch_desc,
           batch_lens, code_emb, code_emb2, desc_emb, Wq, bq, Wk, bk, Wv, bv,
           Wskip, bskip, W2, b2, W3, b3, Wg, bg):
    f32 = jnp.float32

    bd = batch_desc.astype(jnp.int32)
    pad = (-LD) % 16
    bd_p = jnp.pad(bd, ((0, 0), (0, pad)))
    de = desc_emb[bd_p]
    h_n = masked_mean_pallas(de, bd_p, B)

    xt = x.astype(jnp.int32)
    se = code_emb2[xt]
    stmt = masked_mean_pallas(se, xt, 400)

    mt = mini_x.astype(jnp.int32)
    me = code_emb[mt]
    mstmt = masked_mean_pallas(me, mt, 400)

    Wcat = jnp.concatenate([Wq, Wk, Wv, Wskip], axis=1)
    bcat = jnp.concatenate([bq, bk, bv, bskip], axis=0)
    proj = matmul_pallas(mstmt, Wcat, 2000) + bcat[None, :]
    q = proj[:, 0 * D:1 * D]
    kv = proj[:, 1 * D:3 * D]
    skip = proj[:, 3 * D:4 * D]

    src, dst = mini_edge[0], mini_edge[1]
    kv_s = kv[src]
    k_s = kv_s[:, :D].reshape(EM, HEADS, DH)
    v_s = kv_s[:, D:].reshape(EM, HEADS, DH)
    q_d = q[dst].reshape(EM, HEADS, DH)
    alpha = jnp.sum(q_d * k_s, axis=-1) * (1.0 / jnp.sqrt(float(DH)))
    amax = jax.ops.segment_max(alpha, dst, num_segments=NM)
    amax = jnp.where(jnp.isfinite(amax), amax, 0.0)
    ex = jnp.exp(alpha - amax[dst])
    den = jax.ops.segment_sum(ex, dst, num_segments=NM)
    attn = ex / jnp.maximum(den[dst], 1e-16)
    weighted = (v_s * attn[..., None]).reshape(EM, HEADS * DH)
    agg = jax.ops.segment_sum(weighted, dst, num_segments=NM)
    mh = agg + skip

    Wg_p = jnp.pad(Wg, ((0, 0), (0, D - 1)))
    gate_m = matmul_pallas(mh, Wg_p, 2000)[:, 0] + bg[0]
    mfr = _segment_softmax_sum(gate_m, mh, mini_x_batch, N)
    mfr = (mfr + stmt) * 0.5

    row, col, dinv = _gcn_norm(edge_index, N, f32)
    dinv_c = dinv[:, None]

    agg1 = dinv_c * jax.ops.segment_sum((mfr * dinv_c)[row], col,
                                        num_segments=N)
    s1 = matmul_pallas(agg1, W2, 2000)
    hx2 = matmul_pallas(s1, W3, 2000, bias_in=b2)
    s2 = dinv_c * jax.ops.segment_sum((hx2 * dinv_c)[row], col,
                                      num_segments=N)
    h2 = s2 + b3[None, :]

    gate_h = matmul_pallas(h2, Wg_p, 2000)[:, 0] + bg[0]
    fr = _segment_softmax_sum(gate_h, h2, batch, B)

    return cosine_pallas(fr, h_n)
